# Initial kernel scaffold; baseline (speedup 1.0000x reference)
import jax
import jax.numpy as jnp
from jax.experimental import pallas as pl


def kernel(bsz, clip_sim, boundary, control_embedding):
    raise NotImplementedError("write your pallas kernel here")


# speedup vs baseline: 9.0951x; 9.0951x over previous
import functools

import jax
import jax.numpy as jnp
from jax import lax
from jax.experimental import pallas as pl
from jax.experimental.pallas import tpu as pltpu
from jax.experimental.pallas import tpu_sc as plsc

_LANES = 16


@functools.cache
def _make_sc_kernel(B, D, NB, bpw, chunk):
    n_chunks = bpw // chunk
    mesh = plsc.VectorSubcoreMesh(core_axis_name="c", subcore_axis_name="s")

    @functools.partial(
        pl.kernel,
        out_type=jax.ShapeDtypeStruct((B, D), jnp.float32),
        mesh=mesh,
        compiler_params=pltpu.CompilerParams(needs_layout_passes=False),
        scratch_types=[
            pltpu.VMEM((NB,), jnp.float32),
            pltpu.VMEM((bpw,), jnp.float32),
            pltpu.VMEM((bpw,), jnp.int32),
            pltpu.VMEM((2, chunk, D), jnp.float32),
            pltpu.SemaphoreType.DMA,
            pltpu.SemaphoreType.DMA,
            pltpu.SemaphoreType.DMA,
            pltpu.SemaphoreType.DMA,
        ],
    )
    def k(clip_hbm, bnd_hbm, table_hbm, out_hbm,
          bnd_v, clip_v, idx_v, rows_v, gs0, gs1, ws0, ws1):
        nc = 2
        wid = lax.axis_index("s") * nc + lax.axis_index("c")
        base = wid * bpw
        gsem = (gs0, gs1)
        wsem = (ws0, ws1)

        pltpu.sync_copy(bnd_hbm, bnd_v)
        pltpu.sync_copy(clip_hbm.at[pl.ds(base, bpw)], clip_v)

        def bucketize(i, carry):
            x = clip_v[pl.ds(i * _LANES, _LANES)]
            pos = jnp.zeros((_LANES,), jnp.int32)
            for bit in (128, 64, 32, 16, 8, 4, 2, 1):
                probe = pos + bit
                vals = plsc.load_gather(bnd_v, [probe - 1])
                pos = jnp.where(vals <= x, probe, pos)
            idx_v[pl.ds(i * _LANES, _LANES)] = pos
            return carry

        lax.fori_loop(0, bpw // _LANES, bucketize, 0)

        def gather_desc(c):
            buf = c % 2
            return pltpu.make_async_copy(
                table_hbm.at[idx_v.at[pl.ds(c * chunk, chunk)]],
                rows_v.at[buf], gsem[buf])

        def write_desc(c):
            buf = c % 2
            return pltpu.make_async_copy(
                rows_v.at[buf], out_hbm.at[pl.ds(base + c * chunk, chunk)],
                wsem[buf])

        gather_desc(0).start()
        for c in range(n_chunks):
            gather_desc(c).wait()
            write_desc(c).start()
            if c + 1 < n_chunks:
                if c >= 1:
                    write_desc(c - 1).wait()
                gather_desc(c + 1).start()
        write_desc(n_chunks - 2).wait()
        write_desc(n_chunks - 1).wait()

    return k


def kernel(bsz, clip_sim, boundary, control_embedding):
    B = clip_sim.shape[0]
    D = control_embedding.shape[1]
    clip = clip_sim.reshape(B)
    bnd = jnp.concatenate([boundary, jnp.full((1,), jnp.inf, jnp.float32)])
    nw = 32
    bpw = B // nw
    k = _make_sc_kernel(B, D, bnd.shape[0], bpw, 32)
    return k(clip, bnd, control_embedding)
